# baseline (speedup 1.0000x reference)
import jax
import jax.numpy as jnp
from jax.experimental import pallas as pl
from jax.experimental.pallas import tpu as pltpu

_D_MODEL = 4096
_N_EXPERTS = 64
_TILE = 1024


def _router_body(x_ref, w_ref, b_ref, o_ref):
    logits = jax.lax.dot_general(
        x_ref[:], w_ref[:],
        (((1,), (1,)), ((), ())),
        preferred_element_type=jnp.float32,
    ) + b_ref[:]
    m = jnp.max(logits, axis=-1, keepdims=True)
    e = jnp.exp(logits - m)
    o_ref[:] = e / jnp.sum(e, axis=-1, keepdims=True)


def kernel(x, W, b):
    n_tokens = x.shape[0]
    b2 = b.reshape(1, _N_EXPERTS)
    return pl.pallas_call(
        _router_body,
        grid=(n_tokens // _TILE,),
        in_specs=[
            pl.BlockSpec((_TILE, _D_MODEL), lambda i: (i, 0)),
            pl.BlockSpec((_N_EXPERTS, _D_MODEL), lambda i: (0, 0)),
            pl.BlockSpec((1, _N_EXPERTS), lambda i: (0, 0)),
        ],
        out_specs=pl.BlockSpec((_TILE, _N_EXPERTS), lambda i: (i, 0)),
        out_shape=jax.ShapeDtypeStruct((n_tokens, _N_EXPERTS), jnp.float32),
        compiler_params=pltpu.CompilerParams(
            dimension_semantics=("parallel",),
        ),
    )(x, W, b2)
